# Initial kernel scaffold; baseline (speedup 1.0000x reference)
import jax
import jax.numpy as jnp
from jax.experimental import pallas as pl


def kernel(x, W):
    raise NotImplementedError("write your pallas kernel here")


# speedup vs baseline: 4.0464x; 4.0464x over previous
import jax
import jax.numpy as jnp
from jax.experimental import pallas as pl
from jax.experimental.pallas import tpu as pltpu

_B, _C, _K, _T = 64, 32, 1024, 1024


def _vq_body(x_ref, wt_ref, idx_ref, codes_ref):
    xb = x_ref[0]
    wt = wt_ref[...]
    wn = jnp.sum(wt * wt, axis=0)
    prod = jax.lax.dot_general(
        wt, xb, dimension_numbers=(((0,), (0,)), ((), ())),
        preferred_element_type=jnp.float32)
    scores = wn[:, None] - 2.0 * prod
    idx = jnp.argmin(scores, axis=0).astype(jnp.int32)
    idx_ref[0, 0] = idx
    onehot = (jax.lax.broadcasted_iota(jnp.int32, (_K, _T), 0)
              == idx[None, :]).astype(jnp.float32)
    codes_ref[0] = jax.lax.dot_general(
        wt, onehot, dimension_numbers=(((1,), (0,)), ((), ())),
        preferred_element_type=jnp.float32)


def kernel(x, W):
    xr = x.reshape(_B, _C, _T)
    wt = W.T
    idx3, codes3 = pl.pallas_call(
        _vq_body,
        grid=(_B,),
        in_specs=[
            pl.BlockSpec((1, _C, _T), lambda b: (b, 0, 0)),
            pl.BlockSpec((_C, _K), lambda b: (0, 0)),
        ],
        out_specs=[
            pl.BlockSpec((1, 1, _T), lambda b: (b, 0, 0)),
            pl.BlockSpec((1, _C, _T), lambda b: (b, 0, 0)),
        ],
        out_shape=[
            jax.ShapeDtypeStruct((_B, 1, _T), jnp.int32),
            jax.ShapeDtypeStruct((_B, _C, _T), jnp.float32),
        ],
    )(xr, wt)
    codes = codes3.reshape(x.shape)
    indices = idx3.reshape(_B, 32, 32)
    return codes, indices
